# baseline (speedup 1.0000x reference)
import functools

import jax
import jax.numpy as jnp
from jax import lax
from jax.experimental import pallas as pl
from jax.experimental.pallas import tpu as pltpu
from jax.experimental.pallas import tpu_sc as plsc

_C = 8
_T = 16
_NC = 2
_NS = 16
_CH = 32


def _sc_signs_body(F, x_hbm, ab_hbm, sgn_hbm, xv, abv, sv):
    B = x_hbm.shape[0] // F
    TC = _T * _C
    rows_per = B // (_NC * _NS)
    wid = lax.axis_index("s") * _NC + lax.axis_index("c")
    base = wid * rows_per
    pltpu.sync_copy(ab_hbm, abv)

    def chunk_body(ch, carry):
        row0 = base + ch * _CH
        pltpu.sync_copy(x_hbm.at[pl.ds(row0 * F, _CH * F)], xv)

        def row_body(b, carry2):
            boff = jnp.full((16,), b * F, jnp.int32)
            for c in range(_C):
                ia = abv[pl.ds(c * 16, 16)] + boff
                ib = abv[pl.ds(TC + c * 16, 16)] + boff
                xa = plsc.load_gather(xv, [ia])
                xb = plsc.load_gather(xv, [ib])
                sv[pl.ds(b * TC + c * 16, 16)] = jnp.where(
                    xa - xb > 0.0, 1.0, -1.0)
            return carry2

        lax.fori_loop(0, _CH, row_body, 0, unroll=2)
        pltpu.sync_copy(sv, sgn_hbm.at[pl.ds(row0 * TC, _CH * TC)])
        return carry

    lax.fori_loop(0, rows_per // _CH, chunk_body, 0)


def _sc_signs(x, ab):
    B, F = x.shape
    mesh = plsc.VectorSubcoreMesh(core_axis_name="c", subcore_axis_name="s")
    out_flat = pl.kernel(
        functools.partial(_sc_signs_body, F),
        mesh=mesh,
        compiler_params=pltpu.CompilerParams(needs_layout_passes=False),
        out_type=jax.ShapeDtypeStruct((B * _T * _C,), jnp.float32),
        scratch_types=[
            pltpu.VMEM((_CH * F,), jnp.float32),
            pltpu.VMEM((2 * _T * _C,), jnp.int32),
            pltpu.VMEM((_CH * _T * _C,), jnp.float32),
        ],
    )(x.reshape(-1), ab)
    return out_flat.reshape(B, _T * _C)


def _tc_body(sgn_ref, p_ref, tab_ref, o_ref):
    sgn = sgn_ref[...].astype(jnp.bfloat16)
    m = jax.lax.dot_general(
        sgn, p_ref[...],
        dimension_numbers=(((1,), (0,)), ((), ())),
        preferred_element_type=jnp.float32)
    w = (m == float(_C)).astype(jnp.bfloat16)
    o_ref[...] = jax.lax.dot_general(
        w, tab_ref[...],
        dimension_numbers=(((1,), (0,)), ((), ())),
        preferred_element_type=jnp.float32)


def kernel(x, table, anchors_a, anchors_b):
    B, F = x.shape
    T, R, D = table.shape
    C = _C
    ab = jnp.concatenate([
        anchors_a.astype(jnp.int32).T.reshape(-1),
        anchors_b.astype(jnp.int32).T.reshape(-1),
    ])
    sgn = _sc_signs(x, ab)

    ct = jnp.arange(C * T, dtype=jnp.int32)
    tr = jnp.arange(T * R, dtype=jnp.int32)
    same_t = (ct[:, None] % T) == (tr[None, :] // R)
    rbit = ((tr[None, :] % R) >> (ct[:, None] // T)) & 1
    P = jnp.where(same_t,
                  jnp.where(rbit == 1, 1.0, -1.0),
                  0.0).astype(jnp.bfloat16)
    tab = table.reshape(T * R, D).astype(jnp.bfloat16)

    BB = 512
    out = pl.pallas_call(
        _tc_body,
        grid=(B // BB,),
        in_specs=[
            pl.BlockSpec((BB, C * T), lambda i: (i, 0)),
            pl.BlockSpec((C * T, T * R), lambda i: (0, 0)),
            pl.BlockSpec((T * R, D), lambda i: (0, 0)),
        ],
        out_specs=pl.BlockSpec((BB, D), lambda i: (i, 0)),
        out_shape=jax.ShapeDtypeStruct((B, D), jnp.float32),
    )(sgn, P, tab)
    return out


# speedup vs baseline: 1.1120x; 1.1120x over previous
import functools

import jax
import jax.numpy as jnp
from jax import lax
from jax.experimental import pallas as pl
from jax.experimental.pallas import tpu as pltpu
from jax.experimental.pallas import tpu_sc as plsc

_C = 8
_T = 16
_NC = 2
_NS = 16
_CH = 32
_NCHUNK = 4


def _sc_signs_body(row_offset, rows_per, x_hbm, ab_hbm, sgn_hbm, xv, abv, sv):
    TC = _T * _C
    wid = lax.axis_index("s") * _NC + lax.axis_index("c")
    base = wid * rows_per
    pltpu.sync_copy(ab_hbm, abv)

    def chunk_body(ch, carry):
        row0 = base + ch * _CH
        pltpu.sync_copy(x_hbm.at[pl.ds(row_offset + row0, _CH)], xv)

        def row_body(b, carry2):
            rowv = jnp.full((16,), b, jnp.int32)
            for c in range(_C):
                ia = abv[pl.ds(c * 16, 16)]
                ib = abv[pl.ds(TC + c * 16, 16)]
                xa = plsc.load_gather(xv, [rowv, ia])
                xb = plsc.load_gather(xv, [rowv, ib])
                sv[b, pl.ds(c * 16, 16)] = jnp.where(
                    xa - xb > 0.0, 1.0, -1.0)
            return carry2

        lax.fori_loop(0, _CH, row_body, 0, unroll=2)
        pltpu.sync_copy(sv, sgn_hbm.at[pl.ds(row0, _CH)])
        return carry

    lax.fori_loop(0, rows_per // _CH, chunk_body, 0)


def _sc_signs(x, ab, row_offset, rows_chunk):
    F = x.shape[1]
    rows_per = rows_chunk // (_NC * _NS)
    mesh = plsc.VectorSubcoreMesh(core_axis_name="c", subcore_axis_name="s")
    return pl.kernel(
        functools.partial(_sc_signs_body, row_offset, rows_per),
        mesh=mesh,
        compiler_params=pltpu.CompilerParams(needs_layout_passes=False),
        out_type=jax.ShapeDtypeStruct((rows_chunk, _T * _C), jnp.float32),
        scratch_types=[
            pltpu.VMEM((_CH, F), jnp.float32),
            pltpu.VMEM((2 * _T * _C,), jnp.int32),
            pltpu.VMEM((_CH, _T * _C), jnp.float32),
        ],
    )(x, ab)


def _tc_body(sgn_ref, p_ref, tab_ref, o_ref):
    sgn = sgn_ref[...].astype(jnp.bfloat16)
    m = jax.lax.dot_general(
        sgn, p_ref[...],
        dimension_numbers=(((1,), (0,)), ((), ())),
        preferred_element_type=jnp.float32)
    w = (m == float(_C)).astype(jnp.bfloat16)
    o_ref[...] = jax.lax.dot_general(
        w, tab_ref[...],
        dimension_numbers=(((1,), (0,)), ((), ())),
        preferred_element_type=jnp.float32)


def _tc_stage(sgn, P, tab, D):
    Bc = sgn.shape[0]
    BB = 512
    return pl.pallas_call(
        _tc_body,
        grid=(Bc // BB,),
        in_specs=[
            pl.BlockSpec((BB, _C * _T), lambda i: (i, 0)),
            pl.BlockSpec((_C * _T, _T * 256), lambda i: (0, 0)),
            pl.BlockSpec((_T * 256, D), lambda i: (0, 0)),
        ],
        out_specs=pl.BlockSpec((BB, D), lambda i: (i, 0)),
        out_shape=jax.ShapeDtypeStruct((Bc, D), jnp.float32),
    )(sgn, P, tab)


def kernel(x, table, anchors_a, anchors_b):
    B, F = x.shape
    T, R, D = table.shape
    C = _C
    ab = jnp.concatenate([
        anchors_a.astype(jnp.int32).T.reshape(-1),
        anchors_b.astype(jnp.int32).T.reshape(-1),
    ])

    ct = jnp.arange(C * T, dtype=jnp.int32)
    tr = jnp.arange(T * R, dtype=jnp.int32)
    same_t = (ct[:, None] % T) == (tr[None, :] // R)
    rbit = ((tr[None, :] % R) >> (ct[:, None] // T)) & 1
    P = jnp.where(same_t,
                  jnp.where(rbit == 1, 1.0, -1.0),
                  0.0).astype(jnp.bfloat16)
    tab = table.reshape(T * R, D).astype(jnp.bfloat16)

    Bc = B // _NCHUNK
    outs = []
    for i in range(_NCHUNK):
        sgn = _sc_signs(x, ab, i * Bc, Bc)
        outs.append(_tc_stage(sgn, P, tab, D))
    return jnp.concatenate(outs, axis=0)


# speedup vs baseline: 1.2998x; 1.1689x over previous
import functools

import jax
import jax.numpy as jnp
from jax import lax
from jax.experimental import pallas as pl
from jax.experimental.pallas import tpu as pltpu
from jax.experimental.pallas import tpu_sc as plsc

_C = 8
_T = 16
_NC = 2
_NS = 16
_CH = 32
_NCHUNK = 4


def _sc_idx_body(row_offset, rows_per, x_hbm, ab_hbm, idx_hbm, xv, abv, sv):
    TC = _T * _C
    wid = lax.axis_index("s") * _NC + lax.axis_index("c")
    base = wid * rows_per
    pltpu.sync_copy(ab_hbm, abv)

    def chunk_body(ch, carry):
        row0 = base + ch * _CH
        pltpu.sync_copy(x_hbm.at[pl.ds(row_offset + row0, _CH)], xv)

        def row_body(b, carry2):
            rowv = jnp.full((16,), b, jnp.int32)
            acc = jnp.zeros((16,), jnp.float32)
            for c in range(_C):
                ia = abv[pl.ds(c * 16, 16)]
                ib = abv[pl.ds(TC + c * 16, 16)]
                xa = plsc.load_gather(xv, [rowv, ia])
                xb = plsc.load_gather(xv, [rowv, ib])
                acc = acc + jnp.where(xa - xb > 0.0,
                                      jnp.float32(1 << c), 0.0)
            sv[b, :] = acc
            return carry2

        lax.fori_loop(0, _CH, row_body, 0, unroll=4)
        pltpu.sync_copy(sv, idx_hbm.at[pl.ds(row0, _CH)])
        return carry

    lax.fori_loop(0, rows_per // _CH, chunk_body, 0)


def _sc_idx(x, ab, row_offset, rows_chunk):
    F = x.shape[1]
    rows_per = rows_chunk // (_NC * _NS)
    mesh = plsc.VectorSubcoreMesh(core_axis_name="c", subcore_axis_name="s")
    return pl.kernel(
        functools.partial(_sc_idx_body, row_offset, rows_per),
        mesh=mesh,
        compiler_params=pltpu.CompilerParams(needs_layout_passes=False),
        out_type=jax.ShapeDtypeStruct((rows_chunk, _T), jnp.float32),
        scratch_types=[
            pltpu.VMEM((_CH, F), jnp.float32),
            pltpu.VMEM((2 * _T * _C,), jnp.int32),
            pltpu.VMEM((_CH, _T), jnp.float32),
        ],
    )(x, ab)


def _tc_body(idx_ref, r_ref, rpat_ref, tab_ref, *rest):
    (o_ref,) = rest[-1:]
    idxb = idx_ref[...].astype(jnp.bfloat16)
    idxw = jax.lax.dot_general(
        idxb, r_ref[...],
        dimension_numbers=(((1,), (0,)), ((), ())),
        preferred_element_type=jnp.float32)
    w = (idxw == rpat_ref[...]).astype(jnp.bfloat16)
    o_ref[...] = jax.lax.dot_general(
        w, tab_ref[...],
        dimension_numbers=(((1,), (0,)), ((), ())),
        preferred_element_type=jnp.float32)


def _tc_stage(idx, R, rpat, tab, out_prev, chunk_id, B):
    Bc = idx.shape[0]
    D = tab.shape[1]
    BB = 1024
    blk0 = chunk_id * (Bc // BB)
    in_specs = [
        pl.BlockSpec((BB, _T), lambda i: (i, 0)),
        pl.BlockSpec((_T, _T * 256), lambda i: (0, 0)),
        pl.BlockSpec((1, _T * 256), lambda i: (0, 0)),
        pl.BlockSpec((_T * 256, D), lambda i: (0, 0)),
    ]
    args = [idx, R, rpat, tab]
    aliases = {}
    if out_prev is not None:
        in_specs.append(pl.BlockSpec(memory_space=pl.ANY))
        args.append(out_prev)
        aliases = {4: 0}
    return pl.pallas_call(
        _tc_body,
        grid=(Bc // BB,),
        in_specs=in_specs,
        out_specs=pl.BlockSpec((BB, D), lambda i: (blk0 + i, 0)),
        out_shape=jax.ShapeDtypeStruct((B, D), jnp.float32),
        input_output_aliases=aliases,
    )(*args)


def kernel(x, table, anchors_a, anchors_b):
    B, F = x.shape
    T, R_, D = table.shape
    C = _C
    ab = jnp.concatenate([
        anchors_a.astype(jnp.int32).T.reshape(-1),
        anchors_b.astype(jnp.int32).T.reshape(-1),
    ])

    tr = jnp.arange(T * R_, dtype=jnp.int32)
    Rm = ((jnp.arange(T, dtype=jnp.int32)[:, None] == tr[None, :] // R_)
          .astype(jnp.bfloat16))
    rpat = (tr % R_).astype(jnp.float32)[None, :]
    tab = table.reshape(T * R_, D).astype(jnp.bfloat16)

    Bc = B // _NCHUNK
    idxs = [_sc_idx(x, ab, i * Bc, Bc) for i in range(_NCHUNK)]
    out = None
    for i in range(_NCHUNK):
        out = _tc_stage(idxs[i], Rm, rpat, tab, out, i, B)
    return out
